# Initial kernel scaffold; baseline (speedup 1.0000x reference)
import jax
import jax.numpy as jnp
from jax.experimental import pallas as pl


def kernel(x, edge_index, W_self, W_neigh, b):
    raise NotImplementedError("write your pallas kernel here")


# speedup vs baseline: 5.6166x; 5.6166x over previous
import functools

import jax
import jax.numpy as jnp
from jax import lax
from jax.experimental import pallas as pl
from jax.experimental.pallas import tpu as pltpu
from jax.experimental.pallas import tpu_sc as plsc

N_NODES = 10000
N_EDGES = 320000
D_IN = 128
D_OUT = 128

DA = 144
NC = 2
NS = 16
NW = NC * NS
EDGES_PER_TILE = N_EDGES // NW
CHUNK = 80
N_CHUNKS = EDGES_PER_TILE // CHUNK
ROWS_PER_TILE = 632
N_PAD = ROWS_PER_TILE * NS


def _sc_scatter(xa, src, dst, zeros):
    mesh = plsc.VectorSubcoreMesh(
        core_axis_name="c", subcore_axis_name="s", num_cores=NC, num_subcores=NS
    )

    @functools.partial(
        pl.kernel,
        out_type=jax.ShapeDtypeStruct((NC, N_PAD, DA), jnp.float32),
        mesh=mesh,
        scratch_types=[
            pltpu.VMEM((CHUNK,), jnp.int32),
            pltpu.VMEM((CHUNK,), jnp.int32),
            pltpu.VMEM((CHUNK, DA), jnp.float32),
            pltpu.VMEM_SHARED((N_PAD, DA), jnp.float32),
            pltpu.SemaphoreType.DMA,
        ],
        compiler_params=pltpu.CompilerParams(use_tc_tiling_on_sc=False),
    )
    def k(xa_hbm, src_hbm, dst_hbm, zeros_hbm, out_hbm, sidx_v, didx_v, rows_v, acc_sh, sem):
        c = lax.axis_index("c")
        s = lax.axis_index("s")
        w = c * NS + s
        row0 = s * ROWS_PER_TILE

        pltpu.sync_copy(
            zeros_hbm.at[pl.ds(row0, ROWS_PER_TILE)],
            acc_sh.at[pl.ds(row0, ROWS_PER_TILE)],
        )
        plsc.subcore_barrier()

        def body(i, carry):
            base = w * EDGES_PER_TILE + i * CHUNK
            pltpu.sync_copy(src_hbm.at[pl.ds(base, CHUNK)], sidx_v)
            pltpu.sync_copy(dst_hbm.at[pl.ds(base, CHUNK)], didx_v)
            pltpu.async_copy(xa_hbm.at[sidx_v], rows_v, sem).wait()
            pltpu.sync_copy(rows_v, acc_sh.at[didx_v], add=True)
            return carry

        lax.fori_loop(0, N_CHUNKS, body, 0)
        plsc.subcore_barrier()

        pltpu.sync_copy(
            acc_sh.at[pl.ds(row0, ROWS_PER_TILE)],
            out_hbm.at[c, pl.ds(row0, ROWS_PER_TILE)],
        )

    return k(xa, src, dst, zeros)


def _tc_body(x_ref, p_ref, ws_ref, wn_ref, b_ref, o_ref):
    p = p_ref[0] + p_ref[1]
    deg = p[:, D_IN : D_IN + 1]
    mean = p[:, :D_IN] / jnp.maximum(deg, 1.0)
    o_ref[...] = (
        jnp.dot(x_ref[...], ws_ref[...], preferred_element_type=jnp.float32)
        + jnp.dot(mean, wn_ref[...], preferred_element_type=jnp.float32)
        + b_ref[...]
    )


def _tc_dense(x, parts, W_self, W_neigh, b2):
    bm = 1000
    grid = N_NODES // bm
    return pl.pallas_call(
        _tc_body,
        out_shape=jax.ShapeDtypeStruct((N_NODES, D_OUT), jnp.float32),
        grid=(grid,),
        in_specs=[
            pl.BlockSpec((bm, D_IN), lambda i: (i, 0)),
            pl.BlockSpec((NC, bm, DA), lambda i: (0, i, 0)),
            pl.BlockSpec((D_IN, D_OUT), lambda i: (0, 0)),
            pl.BlockSpec((D_IN, D_OUT), lambda i: (0, 0)),
            pl.BlockSpec((1, D_OUT), lambda i: (0, 0)),
        ],
        out_specs=pl.BlockSpec((bm, D_OUT), lambda i: (i, 0)),
    )(x, parts, W_self, W_neigh, b2)


def kernel(x, edge_index, W_self, W_neigh, b):
    src = edge_index[0].astype(jnp.int32)
    dst = edge_index[1].astype(jnp.int32)
    xa = jnp.concatenate(
        [
            x,
            jnp.ones((N_NODES, 1), jnp.float32),
            jnp.zeros((N_NODES, DA - D_IN - 1), jnp.float32),
        ],
        axis=1,
    )
    zeros = jnp.zeros((N_PAD, DA), jnp.float32)
    parts = _sc_scatter(xa, src, dst, zeros)
    return _tc_dense(x, parts, W_self, W_neigh, b.reshape(1, D_OUT))
